# baseline (speedup 1.0000x reference)
import jax
import jax.numpy as jnp
from jax.experimental import pallas as pl
from jax.experimental.pallas import tpu as pltpu

LANE = 128
SUBLANE = 8
MAX_TILE_B = 512


def _round_up(n, m):
    return (n + m - 1) // m * m


def _mlp_body(x_ref, w1_ref, b1_ref, w2_ref, b2_ref, o_ref):
    h = jnp.dot(x_ref[...], w1_ref[...], preferred_element_type=jnp.float32)
    h = jnp.maximum(h + b1_ref[...], 0.0).astype(jnp.bfloat16)
    y = jnp.dot(h, w2_ref[...], preferred_element_type=jnp.float32)
    o_ref[...] = y + b2_ref[...]


def kernel(x, w1_p, b1_p, w2_p, b2_p):
    B, d_in = x.shape
    d_in_p, h_p = w1_p.shape
    _, d_out_p = w2_p.shape
    d_out = 1024

    tile_b = min(MAX_TILE_B, _round_up(B, SUBLANE))
    b_pad = _round_up(B, tile_b)
    nb = b_pad // tile_b

    if (b_pad, d_in_p) == (B, d_in):
        x_b = x.astype(jnp.bfloat16)
    else:
        x_b = jnp.zeros((b_pad, d_in_p), jnp.bfloat16).at[:B, :d_in].set(
            x.astype(jnp.bfloat16))
    w1_b = w1_p.astype(jnp.bfloat16)
    w2_b = w2_p.astype(jnp.bfloat16)

    flops = 2 * b_pad * (d_in_p * h_p + h_p * d_out_p)
    bytes_accessed = (
        2 * b_pad * d_in_p
        + 2 * (d_in_p * h_p + h_p * d_out_p)
        + 4 * (h_p + d_out_p)
        + 4 * b_pad * d_out_p
    )

    out_p = pl.pallas_call(
        _mlp_body,
        out_shape=jax.ShapeDtypeStruct((b_pad, d_out_p), jnp.float32),
        grid_spec=pltpu.PrefetchScalarGridSpec(
            num_scalar_prefetch=0,
            grid=(nb,),
            in_specs=[
                pl.BlockSpec((tile_b, d_in_p), lambda i: (i, 0)),
                pl.BlockSpec((d_in_p, h_p), lambda i: (0, 0)),
                pl.BlockSpec((1, h_p), lambda i: (0, 0)),
                pl.BlockSpec((h_p, d_out_p), lambda i: (0, 0)),
                pl.BlockSpec((1, d_out_p), lambda i: (0, 0)),
            ],
            out_specs=pl.BlockSpec((tile_b, d_out_p), lambda i: (i, 0)),
        ),
        compiler_params=pltpu.CompilerParams(
            dimension_semantics=("parallel",),
        ),
        cost_estimate=pl.CostEstimate(
            flops=flops, transcendentals=0, bytes_accessed=bytes_accessed
        ),
    )(x_b, w1_b, b1_p, w2_b, b2_p)

    return out_p[:B, :d_out]


# speedup vs baseline: 1.1600x; 1.1600x over previous
import jax
import jax.numpy as jnp
from jax.experimental import pallas as pl
from jax.experimental.pallas import tpu as pltpu

LANE = 128
SUBLANE = 8
MAX_TILE_B = 512
NUM_CORES = 2


def _round_up(n, m):
    return (n + m - 1) // m * m


def _mlp_body(x_ref, w1_ref, b1_ref, w2_ref, b2_ref, o_ref, w1b, w2b):
    @pl.when(pl.program_id(1) == 0)
    def _cast_weights():
        w1b[...] = w1_ref[...].astype(jnp.bfloat16)
        w2b[...] = w2_ref[...].astype(jnp.bfloat16)

    xb = x_ref[...].astype(jnp.bfloat16)
    h = jnp.dot(xb, w1b[...], preferred_element_type=jnp.float32)
    h = jnp.maximum(h + b1_ref[...], 0.0).astype(jnp.bfloat16)
    y = jnp.dot(h, w2b[...], preferred_element_type=jnp.float32)
    o_ref[...] = y + b2_ref[...]


def kernel(x, w1_p, b1_p, w2_p, b2_p):
    B, d_in = x.shape
    d_in_p, h_p = w1_p.shape
    _, d_out_p = w2_p.shape
    d_out = 1024

    tile_b = min(MAX_TILE_B, _round_up(B, SUBLANE))
    b_pad = _round_up(B, tile_b)
    nb = b_pad // tile_b
    if nb % NUM_CORES == 0:
        nc, nj = NUM_CORES, nb // NUM_CORES
    else:
        nc, nj = 1, nb

    if (b_pad, d_in_p) == (B, d_in):
        x_p = x
    else:
        x_p = jnp.zeros((b_pad, d_in_p), x.dtype).at[:B, :d_in].set(x)

    flops = 2 * b_pad * (d_in_p * h_p + h_p * d_out_p)
    bytes_accessed = 4 * (
        b_pad * d_in_p
        + d_in_p * h_p + h_p
        + h_p * d_out_p + d_out_p
        + b_pad * d_out_p
    )

    out_p = pl.pallas_call(
        _mlp_body,
        out_shape=jax.ShapeDtypeStruct((b_pad, d_out_p), jnp.float32),
        grid_spec=pltpu.PrefetchScalarGridSpec(
            num_scalar_prefetch=0,
            grid=(nc, nj),
            in_specs=[
                pl.BlockSpec((tile_b, d_in_p), lambda c, j: (c * nj + j, 0)),
                pl.BlockSpec((d_in_p, h_p), lambda c, j: (0, 0)),
                pl.BlockSpec((1, h_p), lambda c, j: (0, 0)),
                pl.BlockSpec((h_p, d_out_p), lambda c, j: (0, 0)),
                pl.BlockSpec((1, d_out_p), lambda c, j: (0, 0)),
            ],
            out_specs=pl.BlockSpec((tile_b, d_out_p), lambda c, j: (c * nj + j, 0)),
            scratch_shapes=[
                pltpu.VMEM((d_in_p, h_p), jnp.bfloat16),
                pltpu.VMEM((h_p, d_out_p), jnp.bfloat16),
            ],
        ),
        compiler_params=pltpu.CompilerParams(
            dimension_semantics=("parallel", "arbitrary"),
            vmem_limit_bytes=112 * 1024 * 1024,
        ),
        cost_estimate=pl.CostEstimate(
            flops=flops, transcendentals=0, bytes_accessed=bytes_accessed
        ),
    )(x_p, w1_p, b1_p, w2_p, b2_p)

    return out_p[:B, :d_out]
